# baseline (speedup 1.0000x reference)
import jax
import jax.numpy as jnp
from jax.experimental import pallas as pl
from jax.experimental.pallas import tpu as pltpu

_BUDGET = 128.0
_SLOPE = 10.0
_NBUF = 4


def _body(mask_ref, sampler_ref, uniform_ref, x_hbm, probs_ref, nm_out_ref,
          o_hbm, nm_s, bin_, bout, sin, sout):
    B, M = x_hbm.shape[0], x_hbm.shape[1]

    for s in range(_NBUF):
        pltpu.make_async_copy(x_hbm.at[s // M, s % M], bin_.at[s], sin.at[s]).start()

    mask_flat = mask_ref[...]
    so = jnp.broadcast_to(sampler_ref[...], mask_flat.shape)
    prob = jax.nn.softplus(_SLOPE * so) / _SLOPE
    denom = jnp.max((1.0 - mask_flat) * prob, axis=1, keepdims=True)
    prob = prob / denom
    masked = prob * (1.0 - mask_flat)
    sparsity = _BUDGET / mask_flat.shape[1]
    xbar = jnp.mean(masked, axis=1, keepdims=True)
    r = sparsity / xbar
    beta = (1.0 - sparsity) / (1.0 - xbar)
    le = (r <= 1.0).astype(masked.dtype)
    normed = le * masked * r + (1.0 - le) * (1.0 - (1.0 - masked) * beta)
    mprob = jnp.where(mask_flat == 0.0, normed, masked)
    binm = (mprob > uniform_ref[...]).astype(jnp.float32)
    nm = mask_flat + binm
    probs_ref[...] = mprob
    nm_out_ref[...] = nm
    nm_s[...] = nm

    N = B * M

    def _out_copy(i, s):
        b, m = i // M, i % M
        pltpu.make_async_copy(bout.at[s], o_hbm.at[b, m], sout.at[s]).start()

    def _group(g, carry):
        for s in range(_NBUF):
            i = g * _NBUF + s
            b = i // M
            pltpu.make_async_copy(x_hbm.at[b, i % M], bin_.at[s], sin.at[s]).wait()

            @pl.when(g > 0)
            def _():
                bq, mq = (i - _NBUF) // M, (i - _NBUF) % M
                pltpu.make_async_copy(bout.at[s], o_hbm.at[bq, mq], sout.at[s]).wait()

            row = nm_s[pl.ds(b, 1), :]
            bout[s] = bin_[s] * row
            _out_copy(i, s)

            @pl.when(g < (N // _NBUF) - 1)
            def _():
                j = i + _NBUF
                pltpu.make_async_copy(x_hbm.at[j // M, j % M], bin_.at[s], sin.at[s]).start()
        return carry

    jax.lax.fori_loop(0, N // _NBUF, _group, 0)

    for s in range(_NBUF):
        i = N - _NBUF + s
        pltpu.make_async_copy(bout.at[s], o_hbm.at[i // M, i % M], sout.at[s]).wait()


def kernel(mask, kspace, sampler, uniform):
    B, M, H, W, C = kspace.shape
    mask_flat = mask.reshape(B, W)
    x = kspace.transpose(0, 1, 2, 4, 3)

    probs, nm, out = pl.pallas_call(
        _body,
        in_specs=[
            pl.BlockSpec((B, W), lambda: (0, 0)),
            pl.BlockSpec((1, W), lambda: (0, 0)),
            pl.BlockSpec((B, W), lambda: (0, 0)),
            pl.BlockSpec(memory_space=pltpu.MemorySpace.HBM),
        ],
        out_specs=(
            pl.BlockSpec((B, W), lambda: (0, 0)),
            pl.BlockSpec((B, W), lambda: (0, 0)),
            pl.BlockSpec(memory_space=pltpu.MemorySpace.HBM),
        ),
        out_shape=(
            jax.ShapeDtypeStruct((B, W), jnp.float32),
            jax.ShapeDtypeStruct((B, W), jnp.float32),
            jax.ShapeDtypeStruct((B, M, H, C, W), jnp.float32),
        ),
        scratch_shapes=[
            pltpu.VMEM((B, W), jnp.float32),
            pltpu.VMEM((_NBUF, H, C, W), jnp.float32),
            pltpu.VMEM((_NBUF, H, C, W), jnp.float32),
            pltpu.SemaphoreType.DMA((_NBUF,)),
            pltpu.SemaphoreType.DMA((_NBUF,)),
        ],
    )(mask_flat, sampler, uniform, x)

    masked_kspace = out.transpose(0, 1, 2, 4, 3)
    new_mask = nm.reshape(B, 1, 1, W, 1)
    final_prob_mask = probs.reshape(B, 1, 1, W, 1)
    return new_mask, masked_kspace, final_prob_mask
